# baseline (speedup 1.0000x reference)
import functools

import jax
import jax.numpy as jnp
from jax.experimental import pallas as pl
from jax.experimental.pallas import tpu as pltpu

_N = 10000
_E = 320000
_R = 1000
_HEADS = 8


def _mlp3_body(x_ref, w1, b1, w2, b2, w3, b3, o_ref):
    x = x_ref[...]
    t = jnp.maximum(jnp.dot(x, w1[...], preferred_element_type=jnp.float32) + b1[...], 0.0)
    t = jnp.maximum(jnp.dot(t, w2[...], preferred_element_type=jnp.float32) + b2[...], 0.0)
    o_ref[...] = jnp.dot(t, w3[...], preferred_element_type=jnp.float32) + b3[...]


def _mlp3_ln_body(x_ref, w1, b1, w2, b2, w3, b3, g, bl, o_ref):
    x = x_ref[...]
    t = jnp.maximum(jnp.dot(x, w1[...], preferred_element_type=jnp.float32) + b1[...], 0.0)
    t = jnp.maximum(jnp.dot(t, w2[...], preferred_element_type=jnp.float32) + b2[...], 0.0)
    t = jnp.dot(t, w3[...], preferred_element_type=jnp.float32) + b3[...]
    m = jnp.mean(t, axis=-1, keepdims=True)
    v = jnp.mean((t - m) ** 2, axis=-1, keepdims=True)
    o_ref[...] = (t - m) * jax.lax.rsqrt(v + 1e-5) * g[...] + bl[...]


def _full(shape):
    nd = len(shape)
    return pl.BlockSpec(shape, lambda i: (0,) * nd)


def _encode(x, Ws, bs, g, b):
    return pl.pallas_call(
        _mlp3_ln_body,
        grid=(_N // _R,),
        in_specs=[pl.BlockSpec((_R, 128), lambda i: (i, 0)),
                  _full((128, 128)), _full((1, 128)),
                  _full((128, 128)), _full((1, 128)),
                  _full((128, 128)), _full((1, 128)),
                  _full((1, 128)), _full((1, 128))],
        out_specs=pl.BlockSpec((_R, 128), lambda i: (i, 0)),
        out_shape=jax.ShapeDtypeStruct((_N, 128), jnp.float32),
    )(x, Ws[0], bs[0].reshape(1, 128), Ws[1], bs[1].reshape(1, 128),
      Ws[2], bs[2].reshape(1, 128), g.reshape(1, 128), b.reshape(1, 128))


def _decode(x, Ws, bs):
    return pl.pallas_call(
        _mlp3_body,
        grid=(_N // _R,),
        in_specs=[pl.BlockSpec((_R, 128), lambda i: (i, 0)),
                  _full((128, 128)), _full((1, 128)),
                  _full((128, 128)), _full((1, 128)),
                  _full((128, 3)), _full((1, 3))],
        out_specs=pl.BlockSpec((_R, 3), lambda i: (i, 0)),
        out_shape=jax.ShapeDtypeStruct((_N, 3), jnp.float32),
    )(x, Ws[0], bs[0].reshape(1, 128), Ws[1], bs[1].reshape(1, 128),
      Ws[2], bs[2].reshape(1, 3))


def _gat_xla(x, src, dst, p, heads, c, concat):
    n = x.shape[0]
    h = (x @ p['W']).reshape(n, heads, c)
    a = (h * p['att_src']).sum(-1)[src] + (h * p['att_dst']).sum(-1)[dst]
    a = jnp.where(a > 0, a, 0.2 * a)
    amax = jax.ops.segment_max(a, dst, num_segments=n)
    amax = jnp.where(jnp.isfinite(amax), amax, 0.0)
    e = jnp.exp(a - amax[dst])
    denom = jax.ops.segment_sum(e, dst, num_segments=n)
    alpha = e / (denom[dst] + 1e-16)
    out = jax.ops.segment_sum(h[src] * alpha[:, :, None], dst, num_segments=n)
    out = out.reshape(n, heads * c) if concat else jnp.mean(out, axis=1)
    return out + p['bias']


def kernel(x, edge_index, params):
    n = x.shape[0]
    loop = jnp.arange(n, dtype=edge_index.dtype)
    src = jnp.concatenate([edge_index[0], loop])
    dst = jnp.concatenate([edge_index[1], loop])

    h = _encode(x, params['enc_Ws'], params['enc_bs'], params['ln_g'], params['ln_b'])
    ngat = len(params['gat'])
    for l, p in enumerate(params['gat']):
        last = (l == ngat - 1)
        h = _gat_xla(h, src, dst, p, _HEADS, 128 if last else 16, not last)
    return _decode(h, params['dec_Ws'], params['dec_bs'])


# speedup vs baseline: 4.6474x; 4.6474x over previous
import functools

import jax
import jax.numpy as jnp
from jax import lax
from jax.experimental import pallas as pl
from jax.experimental.pallas import tpu as pltpu
from jax.experimental.pallas import tpu_sc as plsc

_N = 10000
_E = 320000
_R = 1000
_TILES = 32
_K = 128
_ETT = _E // _TILES
_CH = (_ETT + _K - 1) // _K
_ETP = _CH * _K
_HALF = 1024
_NPASS = 10
_SLAB = 1152
_SLABT = _SLAB // 16


def _mlp3_body(x_ref, w1, b1, w2, b2, w3, b3, o_ref):
    x = x_ref[...]
    t = jnp.maximum(jnp.dot(x, w1[...], preferred_element_type=jnp.float32) + b1[...], 0.0)
    t = jnp.maximum(jnp.dot(t, w2[...], preferred_element_type=jnp.float32) + b2[...], 0.0)
    o_ref[...] = jnp.dot(t, w3[...], preferred_element_type=jnp.float32) + b3[...]


def _mlp3_ln_body(x_ref, w1, b1, w2, b2, w3, b3, g, bl, o_ref):
    x = x_ref[...]
    t = jnp.maximum(jnp.dot(x, w1[...], preferred_element_type=jnp.float32) + b1[...], 0.0)
    t = jnp.maximum(jnp.dot(t, w2[...], preferred_element_type=jnp.float32) + b2[...], 0.0)
    t = jnp.dot(t, w3[...], preferred_element_type=jnp.float32) + b3[...]
    m = jnp.mean(t, axis=-1, keepdims=True)
    v = jnp.mean((t - m) ** 2, axis=-1, keepdims=True)
    o_ref[...] = (t - m) * jax.lax.rsqrt(v + 1e-5) * g[...] + bl[...]


def _full(shape):
    nd = len(shape)
    return pl.BlockSpec(shape, lambda i: (0,) * nd)


def _encode(x, Ws, bs, g, b):
    return pl.pallas_call(
        _mlp3_ln_body,
        grid=(_N // _R,),
        in_specs=[pl.BlockSpec((_R, 128), lambda i: (i, 0)),
                  _full((128, 128)), _full((1, 128)),
                  _full((128, 128)), _full((1, 128)),
                  _full((128, 128)), _full((1, 128)),
                  _full((1, 128)), _full((1, 128))],
        out_specs=pl.BlockSpec((_R, 128), lambda i: (i, 0)),
        out_shape=jax.ShapeDtypeStruct((_N, 128), jnp.float32),
    )(x, Ws[0], bs[0].reshape(1, 128), Ws[1], bs[1].reshape(1, 128),
      Ws[2], bs[2].reshape(1, 128), g.reshape(1, 128), b.reshape(1, 128))


def _decode(x, Ws, bs):
    return pl.pallas_call(
        _mlp3_body,
        grid=(_N // _R,),
        in_specs=[pl.BlockSpec((_R, 128), lambda i: (i, 0)),
                  _full((128, 128)), _full((1, 128)),
                  _full((128, 128)), _full((1, 128)),
                  _full((128, 3)), _full((1, 3))],
        out_specs=pl.BlockSpec((_R, 3), lambda i: (i, 0)),
        out_shape=jax.ShapeDtypeStruct((_N, 3), jnp.float32),
    )(x, Ws[0], bs[0].reshape(1, 128), Ws[1], bs[1].reshape(1, 128),
      Ws[2], bs[2].reshape(1, 3))


def _prep_body(x_ref, w, saf, daf, s16, hout, aso, ado, ato, mso, mdo):
    i = pl.program_id(0)
    h = jnp.dot(x_ref[...], w[...], preferred_element_type=jnp.float32)
    hout[...] = h
    a_s = jnp.dot(h * saf[...], s16[...], preferred_element_type=jnp.float32)
    a_d = jnp.dot(h * daf[...], s16[...], preferred_element_type=jnp.float32)
    aso[...] = a_s
    ado[...] = a_d
    ato[...] = jnp.concatenate(
        [a_s, a_d, jnp.zeros((a_s.shape[0], 96), jnp.float32)], axis=1)

    @pl.when(i == 0)
    def _():
        mso[...] = jnp.full((1, 16), -1e30, jnp.float32)
        mdo[...] = jnp.full((1, 16), -1e30, jnp.float32)

    mso[...] = jnp.maximum(mso[...], jnp.max(a_s, axis=0, keepdims=True))
    mdo[...] = jnp.maximum(mdo[...], jnp.max(a_d, axis=0, keepdims=True))


def _gat_prep(x, W, saf, daf, s16):
    return pl.pallas_call(
        _prep_body,
        grid=(_N // _R,),
        in_specs=[pl.BlockSpec((_R, 128), lambda i: (i, 0)),
                  _full((128, 128)), _full((1, 128)), _full((1, 128)),
                  _full((128, 16))],
        out_specs=[pl.BlockSpec((_R, 128), lambda i: (i, 0)),
                   pl.BlockSpec((_R, 16), lambda i: (i, 0)),
                   pl.BlockSpec((_R, 16), lambda i: (i, 0)),
                   pl.BlockSpec((_R, 128), lambda i: (i, 0)),
                   _full((1, 16)), _full((1, 16))],
        out_shape=[jax.ShapeDtypeStruct((_N, 128), jnp.float32),
                   jax.ShapeDtypeStruct((_N, 16), jnp.float32),
                   jax.ShapeDtypeStruct((_N, 16), jnp.float32),
                   jax.ShapeDtypeStruct((_N, 128), jnp.float32),
                   jax.ShapeDtypeStruct((1, 16), jnp.float32),
                   jax.ShapeDtypeStruct((1, 16), jnp.float32)],
    )(x, W, saf, daf, s16)


def _fin_body(acc0, acc1, den0, den1, h, a_s, a_d, mv, bias, r16, xout):
    av = a_s[...] + a_d[...]
    av = jnp.where(av > 0.0, av, 0.2 * av)
    w16 = jnp.exp(av - mv[...])
    den16 = den0[...] + den1[...] + w16
    num = acc0[...] + acc1[...] + jnp.dot(w16, r16[...], preferred_element_type=jnp.float32) * h[...]
    den128 = jnp.dot(den16, r16[...], preferred_element_type=jnp.float32) + 1e-16
    xout[...] = num / den128 + bias[...]


def _gat_fin(acc0, acc1, den0, den1, h, a_s, a_d, mvec, bias, r16):
    blk = lambda w: pl.BlockSpec((_R, w), lambda i: (i, 0))
    return pl.pallas_call(
        _fin_body,
        grid=(_N // _R,),
        in_specs=[blk(128), blk(128), blk(16), blk(16), blk(128), blk(16), blk(16),
                  _full((1, 16)), _full((1, 128)), _full((16, 128))],
        out_specs=blk(128),
        out_shape=jax.ShapeDtypeStruct((_N, 128), jnp.float32),
    )(acc0, acc1, den0, den1, h, a_s, a_d, mvec, bias, r16)


def _sc_edge_body(h_hbm, at_hbm, mv_hbm, sp_hbm, dg_hbm, bnd_hbm,
                  acc_hbm, den_hbm,
                  sidx, dgidx, didxb, bb, hbuf, asb, adb, wbuf, obuf, mvb,
                  accS, denS, sem0, sem1, sem2):
    c = lax.axis_index("c")
    s = lax.axis_index("s")
    t = c * 16 + s

    base = s * _SLABT
    pltpu.sync_copy(sp_hbm.at[t], sidx)
    pltpu.sync_copy(dg_hbm.at[t], dgidx)
    pltpu.sync_copy(bnd_hbm.at[t], bb)
    pltpu.sync_copy(mv_hbm, mvb)
    mvv = mvb[0, :]
    dumpv = jnp.full((16,), _HALF, jnp.int32) + t

    for p in range(_NPASS):
        lo = p * _HALF
        def zrow(r, _):
            for j in range(8):
                obuf[r, pl.ds(j * 16, 16)] = jnp.zeros((16,), jnp.float32)
            wbuf[r, :] = jnp.zeros((16,), jnp.float32)
            return 0
        lax.fori_loop(0, _SLABT, zrow, 0)
        pltpu.sync_copy(obuf.at[pl.ds(0, _SLABT)], accS.at[pl.ds(base, _SLABT)])
        pltpu.sync_copy(wbuf.at[pl.ds(0, _SLABT)], denS.at[pl.ds(base, _SLABT)])
        plsc.subcore_barrier()

        brow = bb[p, :]
        c0 = brow[0]
        c1 = brow[1]

        def chunk(ci, _):
            g1 = pltpu.async_copy(h_hbm.at[sidx.at[ci]], hbuf, sem0)
            g2 = pltpu.async_copy(at_hbm.at[sidx.at[ci]], asb, sem1)
            g3 = pltpu.async_copy(at_hbm.at[dgidx.at[ci]], adb, sem2)
            for j in range(8):
                dv = dgidx[ci, pl.ds(j * 16, 16)]
                ev = lax.iota(jnp.int32, 16) + (ci * _K + j * 16)
                ok = (dv >= lo) & (dv < lo + _HALF) & (ev < _ETT)
                didxb[pl.ds(j * 16, 16)] = jnp.where(ok, dv - lo, dumpv)
            g1.wait()
            g2.wait()
            g3.wait()

            def edge(e, _):
                av = asb[e, pl.ds(0, 16)] + adb[e, pl.ds(16, 16)]
                av = jnp.where(av > 0.0, av, 0.2 * av)
                w = jnp.exp(av - mvv)
                wbuf[e, :] = w
                for hd in range(8):
                    ws = w[hd]
                    obuf[e, pl.ds(hd * 16, 16)] = hbuf[e, pl.ds(hd * 16, 16)] * ws
                return 0
            lax.fori_loop(0, _K, edge, 0)

            pltpu.sync_copy(obuf, accS.at[didxb], add=True)
            pltpu.sync_copy(wbuf, denS.at[didxb], add=True)
            return 0
        lax.fori_loop(c0, c1, chunk, 0)

        plsc.subcore_barrier()
        pltpu.sync_copy(accS.at[pl.ds(base, _SLABT)],
                        acc_hbm.at[c, p, pl.ds(base, _SLABT)])
        pltpu.sync_copy(denS.at[pl.ds(base, _SLABT)],
                        den_hbm.at[c, p, pl.ds(base, _SLABT)])
        plsc.subcore_barrier()


def _sc_edge_pass(h, atab, mvec, sp, dg, bnd):
    mesh = plsc.VectorSubcoreMesh(core_axis_name="c", subcore_axis_name="s")
    f = functools.partial(
        pl.kernel,
        mesh=mesh,
        out_type=[jax.ShapeDtypeStruct((2, _NPASS, _SLAB, 128), jnp.float32),
                  jax.ShapeDtypeStruct((2, _NPASS, _SLAB, 16), jnp.float32)],
        scratch_types=[
            pltpu.VMEM((_CH, _K), jnp.int32),
            pltpu.VMEM((_CH, _K), jnp.int32),
            pltpu.VMEM((_K,), jnp.int32),
            pltpu.VMEM((_NPASS, 16), jnp.int32),
            pltpu.VMEM((_K, 128), jnp.float32),
            pltpu.VMEM((_K, 128), jnp.float32),
            pltpu.VMEM((_K, 128), jnp.float32),
            pltpu.VMEM((_K, 16), jnp.float32),
            pltpu.VMEM((_K, 128), jnp.float32),
            pltpu.VMEM((1, 16), jnp.float32),
            pltpu.VMEM_SHARED((_SLAB, 128), jnp.float32),
            pltpu.VMEM_SHARED((_SLAB, 16), jnp.float32),
            pltpu.SemaphoreType.DMA,
            pltpu.SemaphoreType.DMA,
            pltpu.SemaphoreType.DMA,
        ],
    )(_sc_edge_body)
    return f(h, atab, mvec, sp, dg, bnd)


def _gat_xla(x, src, dst, p, heads, c, concat):
    n = x.shape[0]
    h = (x @ p['W']).reshape(n, heads, c)
    a = (h * p['att_src']).sum(-1)[src] + (h * p['att_dst']).sum(-1)[dst]
    a = jnp.where(a > 0, a, 0.2 * a)
    amax = jax.ops.segment_max(a, dst, num_segments=n)
    amax = jnp.where(jnp.isfinite(amax), amax, 0.0)
    e = jnp.exp(a - amax[dst])
    denom = jax.ops.segment_sum(e, dst, num_segments=n)
    alpha = e / (denom[dst] + 1e-16)
    out = jax.ops.segment_sum(h[src] * alpha[:, :, None], dst, num_segments=n)
    out = out.reshape(n, heads * c) if concat else jnp.mean(out, axis=1)
    return out + p['bias']


def kernel(x, edge_index, params):
    src = edge_index[0]
    dst = edge_index[1]
    order = jnp.argsort(dst)
    ss = src[order].reshape(_ETT, _TILES).T
    dd = dst[order].reshape(_ETT, _TILES).T
    npad = _ETP - _ETT
    srcp = jnp.concatenate(
        [ss, jnp.zeros((_TILES, npad), jnp.int32)], axis=1).reshape(_TILES, _CH, _K)
    ddp = jnp.concatenate(
        [dd, jnp.full((_TILES, npad), _N - 1, jnp.int32)], axis=1)
    dstg = ddp.reshape(_TILES, _CH, _K)
    cuts = jax.vmap(
        lambda row: jnp.searchsorted(row, jnp.arange(_NPASS + 1) * _HALF)
    )(ddp).astype(jnp.int32)
    cc0 = cuts[:, :-1] // _K
    cc1 = (cuts[:, 1:] + _K - 1) // _K
    bnd = jnp.concatenate(
        [cc0[:, :, None], cc1[:, :, None],
         jnp.zeros((_TILES, _NPASS, 14), jnp.int32)], axis=2)

    s16 = (jnp.arange(128)[:, None] // 16 == jnp.arange(16)[None, :]).astype(jnp.float32)
    r16 = ((jnp.arange(16)[:, None] == jnp.arange(128)[None, :] // 16)
           & (jnp.arange(16)[:, None] < 8)).astype(jnp.float32)

    h = _encode(x, params['enc_Ws'], params['enc_bs'], params['ln_g'], params['ln_b'])

    for l in range(4):
        p = params['gat'][l]
        hh, a_s, a_d, atab, ms, md = _gat_prep(
            h, p['W'], p['att_src'].reshape(1, 128), p['att_dst'].reshape(1, 128), s16)
        m8 = ms[0, :8] + md[0, :8]
        m8 = jnp.where(m8 > 0.0, m8, 0.2 * m8)
        mvec = jnp.concatenate([m8, jnp.full((8,), 1e30, jnp.float32)]).reshape(1, 16)
        acc, den = _sc_edge_pass(hh, atab, mvec, srcp, dstg, bnd)
        lastn = _N - (_NPASS - 1) * _HALF
        cat = lambda a: jnp.concatenate(
            [a[q, :(_HALF if q < _NPASS - 1 else lastn)] for q in range(_NPASS)], axis=0)
        acc0, acc1 = cat(acc[0]), cat(acc[1])
        den0, den1 = cat(den[0]), cat(den[1])
        h = _gat_fin(acc0, acc1, den0, den1,
                     hh, a_s, a_d, mvec, p['bias'].reshape(1, 128), r16)

    loop = jnp.arange(_N, dtype=edge_index.dtype)
    srcf = jnp.concatenate([src, loop])
    dstf = jnp.concatenate([dst, loop])
    h = _gat_xla(h, srcf, dstf, params['gat'][4], 8, 128, False)

    return _decode(h, params['dec_Ws'], params['dec_bs'])


# speedup vs baseline: 35.3355x; 7.6032x over previous
import functools

import jax
import jax.numpy as jnp
from jax import lax
from jax.experimental import pallas as pl
from jax.experimental.pallas import tpu as pltpu
from jax.experimental.pallas import tpu_sc as plsc

_N = 10000
_E = 320000
_R = 1000
_TILES = 32
_K = 128
_ETT = _E // _TILES
_CH = (_ETT + _K - 1) // _K
_ETP = _CH * _K
_HALF = 1024
_NPASS = 10
_SLAB = 1152
_SLABT = _SLAB // 16
_K4 = 32
_CH4 = _ETP // _K4


def _mlp3_body(x_ref, w1, b1, w2, b2, w3, b3, o_ref):
    x = x_ref[...]
    t = jnp.maximum(jnp.dot(x, w1[...], preferred_element_type=jnp.float32) + b1[...], 0.0)
    t = jnp.maximum(jnp.dot(t, w2[...], preferred_element_type=jnp.float32) + b2[...], 0.0)
    o_ref[...] = jnp.dot(t, w3[...], preferred_element_type=jnp.float32) + b3[...]


def _mlp3_ln_body(x_ref, w1, b1, w2, b2, w3, b3, g, bl, o_ref):
    x = x_ref[...]
    t = jnp.maximum(jnp.dot(x, w1[...], preferred_element_type=jnp.float32) + b1[...], 0.0)
    t = jnp.maximum(jnp.dot(t, w2[...], preferred_element_type=jnp.float32) + b2[...], 0.0)
    t = jnp.dot(t, w3[...], preferred_element_type=jnp.float32) + b3[...]
    m = jnp.mean(t, axis=-1, keepdims=True)
    v = jnp.mean((t - m) ** 2, axis=-1, keepdims=True)
    o_ref[...] = (t - m) * jax.lax.rsqrt(v + 1e-5) * g[...] + bl[...]


def _full(shape):
    nd = len(shape)
    return pl.BlockSpec(shape, lambda i: (0,) * nd)


def _encode(x, Ws, bs, g, b):
    return pl.pallas_call(
        _mlp3_ln_body,
        grid=(_N // _R,),
        in_specs=[pl.BlockSpec((_R, 128), lambda i: (i, 0)),
                  _full((128, 128)), _full((1, 128)),
                  _full((128, 128)), _full((1, 128)),
                  _full((128, 128)), _full((1, 128)),
                  _full((1, 128)), _full((1, 128))],
        out_specs=pl.BlockSpec((_R, 128), lambda i: (i, 0)),
        out_shape=jax.ShapeDtypeStruct((_N, 128), jnp.float32),
    )(x, Ws[0], bs[0].reshape(1, 128), Ws[1], bs[1].reshape(1, 128),
      Ws[2], bs[2].reshape(1, 128), g.reshape(1, 128), b.reshape(1, 128))


def _decode(x, Ws, bs):
    return pl.pallas_call(
        _mlp3_body,
        grid=(_N // _R,),
        in_specs=[pl.BlockSpec((_R, 128), lambda i: (i, 0)),
                  _full((128, 128)), _full((1, 128)),
                  _full((128, 128)), _full((1, 128)),
                  _full((128, 3)), _full((1, 3))],
        out_specs=pl.BlockSpec((_R, 3), lambda i: (i, 0)),
        out_shape=jax.ShapeDtypeStruct((_N, 3), jnp.float32),
    )(x, Ws[0], bs[0].reshape(1, 128), Ws[1], bs[1].reshape(1, 128),
      Ws[2], bs[2].reshape(1, 3))


def _prep_body(x_ref, w, saf, daf, s16, hout, aso, ado, ato, mso, mdo):
    i = pl.program_id(0)
    h = jnp.dot(x_ref[...], w[...], preferred_element_type=jnp.float32)
    hout[...] = h
    a_s = jnp.dot(h * saf[...], s16[...], preferred_element_type=jnp.float32)
    a_d = jnp.dot(h * daf[...], s16[...], preferred_element_type=jnp.float32)
    aso[...] = a_s
    ado[...] = a_d
    ato[...] = jnp.concatenate(
        [a_s, a_d, jnp.zeros((a_s.shape[0], 96), jnp.float32)], axis=1)

    @pl.when(i == 0)
    def _():
        mso[...] = jnp.full((1, 16), -1e30, jnp.float32)
        mdo[...] = jnp.full((1, 16), -1e30, jnp.float32)

    mso[...] = jnp.maximum(mso[...], jnp.max(a_s, axis=0, keepdims=True))
    mdo[...] = jnp.maximum(mdo[...], jnp.max(a_d, axis=0, keepdims=True))


def _gat_prep(x, W, saf, daf, s16):
    return pl.pallas_call(
        _prep_body,
        grid=(_N // _R,),
        in_specs=[pl.BlockSpec((_R, 128), lambda i: (i, 0)),
                  _full((128, 128)), _full((1, 128)), _full((1, 128)),
                  _full((128, 16))],
        out_specs=[pl.BlockSpec((_R, 128), lambda i: (i, 0)),
                   pl.BlockSpec((_R, 16), lambda i: (i, 0)),
                   pl.BlockSpec((_R, 16), lambda i: (i, 0)),
                   pl.BlockSpec((_R, 128), lambda i: (i, 0)),
                   _full((1, 16)), _full((1, 16))],
        out_shape=[jax.ShapeDtypeStruct((_N, 128), jnp.float32),
                   jax.ShapeDtypeStruct((_N, 16), jnp.float32),
                   jax.ShapeDtypeStruct((_N, 16), jnp.float32),
                   jax.ShapeDtypeStruct((_N, 128), jnp.float32),
                   jax.ShapeDtypeStruct((1, 16), jnp.float32),
                   jax.ShapeDtypeStruct((1, 16), jnp.float32)],
    )(x, W, saf, daf, s16)


def _fin_body(acc0, acc1, den0, den1, h, a_s, a_d, mv, bias, r16, xout):
    av = a_s[...] + a_d[...]
    av = jnp.where(av > 0.0, av, 0.2 * av)
    w16 = jnp.exp(av - mv[...])
    den16 = den0[...] + den1[...] + w16
    num = acc0[...] + acc1[...] + jnp.dot(w16, r16[...], preferred_element_type=jnp.float32) * h[...]
    den128 = jnp.dot(den16, r16[...], preferred_element_type=jnp.float32) + 1e-16
    xout[...] = num / den128 + bias[...]


def _gat_fin(acc0, acc1, den0, den1, h, a_s, a_d, mvec, bias, r16):
    blk = lambda w: pl.BlockSpec((_R, w), lambda i: (i, 0))
    return pl.pallas_call(
        _fin_body,
        grid=(_N // _R,),
        in_specs=[blk(128), blk(128), blk(16), blk(16), blk(128), blk(16), blk(16),
                  _full((1, 16)), _full((1, 128)), _full((16, 128))],
        out_specs=blk(128),
        out_shape=jax.ShapeDtypeStruct((_N, 128), jnp.float32),
    )(acc0, acc1, den0, den1, h, a_s, a_d, mvec, bias, r16)


def _prep4_body(x_ref, w, saf, daf, s8, hout, aso, ado, ato, mso, mdo):
    i = pl.program_id(0)
    h = jnp.dot(x_ref[...], w[...], preferred_element_type=jnp.float32)
    hout[...] = h
    a_s = jnp.dot(h * saf[...], s8[...], preferred_element_type=jnp.float32)
    a_d = jnp.dot(h * daf[...], s8[...], preferred_element_type=jnp.float32)
    aso[...] = a_s
    ado[...] = a_d
    ato[...] = jnp.concatenate(
        [a_s, a_d, jnp.zeros((a_s.shape[0], 96), jnp.float32)], axis=1)

    @pl.when(i == 0)
    def _():
        mso[...] = jnp.full((1, 16), -1e30, jnp.float32)
        mdo[...] = jnp.full((1, 16), -1e30, jnp.float32)

    mso[...] = jnp.maximum(mso[...], jnp.max(a_s, axis=0, keepdims=True))
    mdo[...] = jnp.maximum(mdo[...], jnp.max(a_d, axis=0, keepdims=True))


def _gat_prep4(x, W, saf, daf, s8):
    return pl.pallas_call(
        _prep4_body,
        grid=(_N // _R,),
        in_specs=[pl.BlockSpec((_R, 128), lambda i: (i, 0)),
                  _full((128, 1024)), _full((1, 1024)), _full((1, 1024)),
                  _full((1024, 16))],
        out_specs=[pl.BlockSpec((_R, 1024), lambda i: (i, 0)),
                   pl.BlockSpec((_R, 16), lambda i: (i, 0)),
                   pl.BlockSpec((_R, 16), lambda i: (i, 0)),
                   pl.BlockSpec((_R, 128), lambda i: (i, 0)),
                   _full((1, 16)), _full((1, 16))],
        out_shape=[jax.ShapeDtypeStruct((_N, 1024), jnp.float32),
                   jax.ShapeDtypeStruct((_N, 16), jnp.float32),
                   jax.ShapeDtypeStruct((_N, 16), jnp.float32),
                   jax.ShapeDtypeStruct((_N, 128), jnp.float32),
                   jax.ShapeDtypeStruct((1, 16), jnp.float32),
                   jax.ShapeDtypeStruct((1, 16), jnp.float32)],
    )(x, W, saf, daf, s8)


def _mid4_body(den0, den1, a_s, a_d, mv, h4, r16h, s8sum, rto, sco):
    av = a_s[...] + a_d[...]
    av = jnp.where(av > 0.0, av, 0.2 * av)
    w16 = jnp.exp(av - mv[...])
    r = 1.0 / (den0[...] + den1[...] + w16 + 1e-16)
    rto[...] = jnp.concatenate(
        [r, jnp.zeros((r.shape[0], 112), jnp.float32)], axis=1)
    alpha = jnp.dot(w16 * r, r16h[...], preferred_element_type=jnp.float32)
    sco[...] = jnp.dot(alpha * h4[...], s8sum[...],
                       preferred_element_type=jnp.float32)


def _gat_mid4(den0, den1, a_s, a_d, mvec, h4, r16h, s8sum):
    blk = lambda w: pl.BlockSpec((_R, w), lambda i: (i, 0))
    return pl.pallas_call(
        _mid4_body,
        grid=(_N // _R,),
        in_specs=[blk(16), blk(16), blk(16), blk(16), _full((1, 16)),
                  blk(1024), _full((16, 1024)), _full((1024, 128))],
        out_specs=[blk(128), blk(128)],
        out_shape=[jax.ShapeDtypeStruct((_N, 128), jnp.float32),
                   jax.ShapeDtypeStruct((_N, 128), jnp.float32)],
    )(den0, den1, a_s, a_d, mvec, h4, r16h, s8sum)


def _fin4_body(acc0, acc1, sc, bias, o_ref):
    o_ref[...] = (acc0[...] + acc1[...] + sc[...]) * 0.125 + bias[...]


def _gat_fin4(acc0, acc1, selfc, bias):
    blk = lambda w: pl.BlockSpec((_R, w), lambda i: (i, 0))
    return pl.pallas_call(
        _fin4_body,
        grid=(_N // _R,),
        in_specs=[blk(128), blk(128), blk(128), _full((1, 128))],
        out_specs=blk(128),
        out_shape=jax.ShapeDtypeStruct((_N, 128), jnp.float32),
    )(acc0, acc1, selfc, bias)


def _sc_edge_body(h_hbm, at_hbm, mv_hbm, pi_hbm, bnd_hbm,
                  acc_hbm, den_hbm,
                  pidxv, sidxb, dgb, didxb, bb, hbuf, asb, adb, wbuf, obuf, mvb,
                  accS, denS, sem0, sem1, sem2):
    c = lax.axis_index("c")
    s = lax.axis_index("s")
    t = c * 16 + s

    base = s * _SLABT
    pltpu.sync_copy(pi_hbm.at[t], pidxv)
    pltpu.sync_copy(bnd_hbm.at[t], bb)
    pltpu.sync_copy(mv_hbm, mvb)
    mvv = mvb[0, :]
    dumpv = jnp.full((16,), _HALF, jnp.int32) + t

    for p in range(_NPASS):
        lo = p * _HALF
        def zrow(r, _):
            for j in range(8):
                obuf[r, pl.ds(j * 16, 16)] = jnp.zeros((16,), jnp.float32)
            wbuf[r, :] = jnp.zeros((16,), jnp.float32)
            return 0
        lax.fori_loop(0, min(_SLABT, _K), zrow, 0)
        _nf, _rem = _SLABT // _K, _SLABT % _K
        for q in range(_nf):
            pltpu.sync_copy(obuf, accS.at[pl.ds(base + q * _K, _K)])
            pltpu.sync_copy(wbuf, denS.at[pl.ds(base + q * _K, _K)])
        if _rem:
            pltpu.sync_copy(obuf.at[pl.ds(0, _rem)],
                            accS.at[pl.ds(base + _nf * _K, _rem)])
            pltpu.sync_copy(wbuf.at[pl.ds(0, _rem)],
                            denS.at[pl.ds(base + _nf * _K, _rem)])
        plsc.subcore_barrier()

        brow = bb[p, :]
        c0 = brow[0]
        c1 = brow[1]

        def chunk(ci, _):
            for j in range(_K // 16):
                pv = pidxv[ci, pl.ds(j * 16, 16)]
                sv = jnp.right_shift(pv, 14)
                dv = pv & 16383
                sidxb[pl.ds(j * 16, 16)] = sv
                dgb[pl.ds(j * 16, 16)] = dv
                ev = lax.iota(jnp.int32, 16) + (ci * _K + j * 16)
                ok = (dv >= lo) & (dv < lo + _HALF) & (ev < _ETT)
                didxb[pl.ds(j * 16, 16)] = jnp.where(ok, dv - lo, dumpv)
            g1 = pltpu.async_copy(h_hbm.at[sidxb], hbuf, sem0)
            g2 = pltpu.async_copy(at_hbm.at[sidxb], asb, sem1)
            g3 = pltpu.async_copy(at_hbm.at[dgb], adb, sem2)
            g1.wait()
            g2.wait()
            g3.wait()

            def edge(e, _):
                av = asb[e, pl.ds(0, 16)] + adb[e, pl.ds(16, 16)]
                av = jnp.where(av > 0.0, av, 0.2 * av)
                w = jnp.exp(av - mvv)
                wbuf[e, :] = w
                for hd in range(8):
                    ws = w[hd]
                    obuf[e, pl.ds(hd * 16, 16)] = hbuf[e, pl.ds(hd * 16, 16)] * ws
                return 0
            lax.fori_loop(0, _K, edge, 0)

            pltpu.sync_copy(obuf, accS.at[didxb], add=True)
            pltpu.sync_copy(wbuf, denS.at[didxb], add=True)
            return 0
        lax.fori_loop(c0, c1, chunk, 0)

        plsc.subcore_barrier()
        pltpu.sync_copy(accS.at[pl.ds(base, _SLABT)],
                        acc_hbm.at[c, p, pl.ds(base, _SLABT)])
        pltpu.sync_copy(denS.at[pl.ds(base, _SLABT)],
                        den_hbm.at[c, p, pl.ds(base, _SLABT)])
        plsc.subcore_barrier()


def _sc_edge_pass(h, atab, mvec, pidx, bnd):
    mesh = plsc.VectorSubcoreMesh(core_axis_name="c", subcore_axis_name="s")
    f = functools.partial(
        pl.kernel,
        mesh=mesh,
        out_type=[jax.ShapeDtypeStruct((2, _NPASS, _SLAB, 128), jnp.float32),
                  jax.ShapeDtypeStruct((2, _NPASS, _SLAB, 16), jnp.float32)],
        scratch_types=[
            pltpu.VMEM((_CH, _K), jnp.int32),
            pltpu.VMEM((_K,), jnp.int32),
            pltpu.VMEM((_K,), jnp.int32),
            pltpu.VMEM((_K,), jnp.int32),
            pltpu.VMEM((_NPASS, 16), jnp.int32),
            pltpu.VMEM((_K, 128), jnp.float32),
            pltpu.VMEM((_K, 128), jnp.float32),
            pltpu.VMEM((_K, 128), jnp.float32),
            pltpu.VMEM((_K, 16), jnp.float32),
            pltpu.VMEM((_K, 128), jnp.float32),
            pltpu.VMEM((1, 16), jnp.float32),
            pltpu.VMEM_SHARED((_SLAB, 128), jnp.float32),
            pltpu.VMEM_SHARED((_SLAB, 16), jnp.float32),
            pltpu.SemaphoreType.DMA,
            pltpu.SemaphoreType.DMA,
            pltpu.SemaphoreType.DMA,
        ],
    )(_sc_edge_body)
    return f(h, atab, mvec, pidx, bnd)


def _sc_wden_body(at_hbm, mv_hbm, pi_hbm, bnd_hbm,
                  den_hbm,
                  pidxv, sidxb, dgb, didxb, bb, asb, adb, wbuf, zbuf, mvb,
                  denS, sem1, sem2):
    c = lax.axis_index("c")
    s = lax.axis_index("s")
    t = c * 16 + s
    base = s * _SLABT
    pltpu.sync_copy(pi_hbm.at[t], pidxv)
    pltpu.sync_copy(bnd_hbm.at[t], bb)
    pltpu.sync_copy(mv_hbm, mvb)
    mvv = mvb[0, :]
    dumpv = jnp.full((16,), _HALF, jnp.int32) + t

    def zrow(r, _):
        zbuf[r, :] = jnp.zeros((16,), jnp.float32)
        return 0
    lax.fori_loop(0, _SLABT, zrow, 0)

    for p in range(_NPASS):
        lo = p * _HALF
        pltpu.sync_copy(zbuf, denS.at[pl.ds(base, _SLABT)])
        plsc.subcore_barrier()
        brow = bb[p, :]

        def chunk(ci, _):
            for j in range(_K // 16):
                pv = pidxv[ci, pl.ds(j * 16, 16)]
                sv = jnp.right_shift(pv, 14)
                dv = pv & 16383
                sidxb[pl.ds(j * 16, 16)] = sv
                dgb[pl.ds(j * 16, 16)] = dv
                ev = lax.iota(jnp.int32, 16) + (ci * _K + j * 16)
                ok = (dv >= lo) & (dv < lo + _HALF) & (ev < _ETT)
                didxb[pl.ds(j * 16, 16)] = jnp.where(ok, dv - lo, dumpv)
            g2 = pltpu.async_copy(at_hbm.at[sidxb], asb, sem1)
            g3 = pltpu.async_copy(at_hbm.at[dgb], adb, sem2)
            g2.wait()
            g3.wait()

            def edge(e, _):
                av = asb[e, pl.ds(0, 16)] + adb[e, pl.ds(16, 16)]
                av = jnp.where(av > 0.0, av, 0.2 * av)
                wbuf[e, :] = jnp.exp(av - mvv)
                return 0
            lax.fori_loop(0, _K, edge, 0)

            pltpu.sync_copy(wbuf, denS.at[didxb], add=True)
            return 0
        lax.fori_loop(brow[0], brow[1], chunk, 0)

        plsc.subcore_barrier()
        pltpu.sync_copy(denS.at[pl.ds(base, _SLABT)],
                        den_hbm.at[c, p, pl.ds(base, _SLABT)])
        plsc.subcore_barrier()


def _sc_wden(atab, mvec, pidx, bnd):
    mesh = plsc.VectorSubcoreMesh(core_axis_name="c", subcore_axis_name="s")
    f = functools.partial(
        pl.kernel,
        mesh=mesh,
        out_type=jax.ShapeDtypeStruct((2, _NPASS, _SLAB, 16), jnp.float32),
        scratch_types=[
            pltpu.VMEM((_CH, _K), jnp.int32),
            pltpu.VMEM((_K,), jnp.int32),
            pltpu.VMEM((_K,), jnp.int32),
            pltpu.VMEM((_K,), jnp.int32),
            pltpu.VMEM((_NPASS, 16), jnp.int32),
            pltpu.VMEM((_K, 128), jnp.float32),
            pltpu.VMEM((_K, 128), jnp.float32),
            pltpu.VMEM((_K, 16), jnp.float32),
            pltpu.VMEM((_SLABT, 16), jnp.float32),
            pltpu.VMEM((1, 16), jnp.float32),
            pltpu.VMEM_SHARED((_SLAB, 16), jnp.float32),
            pltpu.SemaphoreType.DMA,
            pltpu.SemaphoreType.DMA,
        ],
    )(_sc_wden_body)
    return f(atab, mvec, pidx, bnd)


def _sc_aggr_body(h4_hbm, rt_hbm, at_hbm, mv_hbm, pi_hbm, bnd_hbm,
                  acc_hbm,
                  pidxv, sidxb, dgb, didxb, bb, h4b, rb, asb, adb, obuf, mvb,
                  accS, sem0, sem1, sem2, sem3):
    c = lax.axis_index("c")
    s = lax.axis_index("s")
    t = c * 16 + s
    base = s * _SLABT
    pltpu.sync_copy(pi_hbm.at[t], pidxv)
    pltpu.sync_copy(bnd_hbm.at[t], bb)
    pltpu.sync_copy(mv_hbm, mvb)
    mvv = mvb[0, :]
    dumpv = jnp.full((16,), _HALF, jnp.int32) + t

    for p in range(_NPASS):
        lo = p * _HALF

        def zrow(r, _):
            for j in range(8):
                obuf[r, pl.ds(j * 16, 16)] = jnp.zeros((16,), jnp.float32)
            return 0
        lax.fori_loop(0, _K4, zrow, 0)
        _nf, _rem = _SLABT // _K4, _SLABT % _K4
        for q in range(_nf):
            pltpu.sync_copy(obuf, accS.at[pl.ds(base + q * _K4, _K4)])
        if _rem:
            pltpu.sync_copy(obuf.at[pl.ds(0, _rem)],
                            accS.at[pl.ds(base + _nf * _K4, _rem)])
        plsc.subcore_barrier()
        brow = bb[p, :]

        def chunk(ci, _):
            for j in range(2):
                pv = pidxv[ci, pl.ds(j * 16, 16)]
                sv = jnp.right_shift(pv, 14)
                dv = pv & 16383
                sidxb[pl.ds(j * 16, 16)] = sv
                dgb[pl.ds(j * 16, 16)] = dv
                ev = lax.iota(jnp.int32, 16) + (ci * _K4 + j * 16)
                ok = (dv >= lo) & (dv < lo + _HALF) & (ev < _ETT)
                didxb[pl.ds(j * 16, 16)] = jnp.where(ok, dv - lo, dumpv)
            g1 = pltpu.async_copy(h4_hbm.at[sidxb], h4b, sem0)
            g2 = pltpu.async_copy(rt_hbm.at[dgb], rb, sem1)
            g3 = pltpu.async_copy(at_hbm.at[sidxb], asb, sem2)
            g4 = pltpu.async_copy(at_hbm.at[dgb], adb, sem3)
            g1.wait()
            g2.wait()
            g3.wait()
            g4.wait()

            def edge(e, _):
                av = asb[e, pl.ds(0, 16)] + adb[e, pl.ds(16, 16)]
                av = jnp.where(av > 0.0, av, 0.2 * av)
                alpha = jnp.exp(av - mvv) * rb[e, pl.ds(0, 16)]
                a = [alpha[hd] for hd in range(8)]
                for j in range(8):
                    v = h4b[e, pl.ds(j * 16, 16)] * a[0]
                    for hd in range(1, 8):
                        v = v + h4b[e, pl.ds(hd * 128 + j * 16, 16)] * a[hd]
                    obuf[e, pl.ds(j * 16, 16)] = v
                return 0
            lax.fori_loop(0, _K4, edge, 0)

            pltpu.sync_copy(obuf, accS.at[didxb], add=True)
            return 0
        lax.fori_loop(brow[0], brow[1], chunk, 0)

        plsc.subcore_barrier()
        pltpu.sync_copy(accS.at[pl.ds(base, _SLABT)],
                        acc_hbm.at[c, p, pl.ds(base, _SLABT)])
        plsc.subcore_barrier()


def _sc_aggr(h4, rtab, atab, mvec, pidx4, bnd4):
    mesh = plsc.VectorSubcoreMesh(core_axis_name="c", subcore_axis_name="s")
    f = functools.partial(
        pl.kernel,
        mesh=mesh,
        out_type=jax.ShapeDtypeStruct((2, _NPASS, _SLAB, 128), jnp.float32),
        scratch_types=[
            pltpu.VMEM((_CH4, _K4), jnp.int32),
            pltpu.VMEM((_K4,), jnp.int32),
            pltpu.VMEM((_K4,), jnp.int32),
            pltpu.VMEM((_K4,), jnp.int32),
            pltpu.VMEM((_NPASS, 16), jnp.int32),
            pltpu.VMEM((_K4, 1024), jnp.float32),
            pltpu.VMEM((_K4, 128), jnp.float32),
            pltpu.VMEM((_K4, 128), jnp.float32),
            pltpu.VMEM((_K4, 128), jnp.float32),
            pltpu.VMEM((_K4, 128), jnp.float32),
            pltpu.VMEM((1, 16), jnp.float32),
            pltpu.VMEM_SHARED((_SLAB, 128), jnp.float32),
            pltpu.SemaphoreType.DMA,
            pltpu.SemaphoreType.DMA,
            pltpu.SemaphoreType.DMA,
            pltpu.SemaphoreType.DMA,
        ],
    )(_sc_aggr_body)
    return f(h4, rtab, atab, mvec, pidx4, bnd4)


def kernel(x, edge_index, params):
    src = edge_index[0]
    dst = edge_index[1]
    order = jnp.argsort(dst)
    ss = src[order].reshape(_ETT, _TILES).T
    dd = dst[order].reshape(_ETT, _TILES).T
    npad = _ETP - _ETT
    ssp = jnp.concatenate([ss, jnp.zeros((_TILES, npad), jnp.int32)], axis=1)
    ddp = jnp.concatenate(
        [dd, jnp.full((_TILES, npad), _N - 1, jnp.int32)], axis=1)
    pidx = (ssp * 16384 + ddp).reshape(_TILES, _CH, _K)
    cuts = jax.vmap(
        lambda row: jnp.searchsorted(row, jnp.arange(_NPASS + 1) * _HALF)
    )(ddp).astype(jnp.int32)
    cc0 = cuts[:, :-1] // _K
    cc1 = (cuts[:, 1:] + _K - 1) // _K
    bnd = jnp.concatenate(
        [cc0[:, :, None], cc1[:, :, None],
         jnp.zeros((_TILES, _NPASS, 14), jnp.int32)], axis=2)

    s16 = (jnp.arange(128)[:, None] // 16 == jnp.arange(16)[None, :]).astype(jnp.float32)
    r16 = ((jnp.arange(16)[:, None] == jnp.arange(128)[None, :] // 16)
           & (jnp.arange(16)[:, None] < 8)).astype(jnp.float32)

    h = _encode(x, params['enc_Ws'], params['enc_bs'], params['ln_g'], params['ln_b'])

    lastn = _N - (_NPASS - 1) * _HALF
    cat = lambda a: jnp.concatenate(
        [a[q, :(_HALF if q < _NPASS - 1 else lastn)] for q in range(_NPASS)], axis=0)

    stacked = jax.tree.map(lambda *x: jnp.stack(x), *params['gat'][:4])

    def _layer(hc, p):
        hh, a_s, a_d, atab, ms, md = _gat_prep(
            hc, p['W'], p['att_src'].reshape(1, 128), p['att_dst'].reshape(1, 128), s16)
        m8 = ms[0, :8] + md[0, :8]
        m8 = jnp.where(m8 > 0.0, m8, 0.2 * m8)
        mvec = jnp.concatenate([m8, jnp.full((8,), 1e30, jnp.float32)]).reshape(1, 16)
        acc, den = _sc_edge_pass(hh, atab, mvec, pidx, bnd)
        hn = _gat_fin(cat(acc[0]), cat(acc[1]), cat(den[0]), cat(den[1]),
                      hh, a_s, a_d, mvec, p['bias'].reshape(1, 128), r16)
        return hn, None

    h, _ = lax.scan(_layer, h, stacked)

    p4 = params['gat'][4]
    s8 = (jnp.arange(1024)[:, None] // 128 == jnp.arange(16)[None, :]).astype(jnp.float32)
    r16h = (jnp.arange(16)[:, None] == jnp.arange(1024)[None, :] // 128).astype(jnp.float32)
    s8sum = (jnp.arange(1024)[:, None] % 128 == jnp.arange(128)[None, :]).astype(jnp.float32)
    hh4, as4, ad4, atab4, ms4, md4 = _gat_prep4(
        h, p4['W'], p4['att_src'].reshape(1, 1024), p4['att_dst'].reshape(1, 1024), s8)
    m84 = ms4[0, :8] + md4[0, :8]
    m84 = jnp.where(m84 > 0.0, m84, 0.2 * m84)
    mvec4 = jnp.concatenate([m84, jnp.full((8,), 1e30, jnp.float32)]).reshape(1, 16)
    den4 = _sc_wden(atab4, mvec4, pidx, bnd)
    lastn = _N - (_NPASS - 1) * _HALF
    cat = lambda a: jnp.concatenate(
        [a[q, :(_HALF if q < _NPASS - 1 else lastn)] for q in range(_NPASS)], axis=0)
    rtab, selfc = _gat_mid4(cat(den4[0]), cat(den4[1]), as4, ad4, mvec4,
                            hh4, r16h, s8sum)
    pidx4 = pidx.reshape(_TILES, _CH4, _K4)
    cc0_4 = cuts[:, :-1] // _K4
    cc1_4 = (cuts[:, 1:] + _K4 - 1) // _K4
    bnd4 = jnp.concatenate(
        [cc0_4[:, :, None], cc1_4[:, :, None],
         jnp.zeros((_TILES, _NPASS, 14), jnp.int32)], axis=2)
    acc4 = _sc_aggr(hh4, rtab, atab4, mvec4, pidx4, bnd4)
    x4 = _gat_fin4(cat(acc4[0]), cat(acc4[1]), selfc, p4['bias'].reshape(1, 128))

    return _decode(x4, params['dec_Ws'], params['dec_bs'])
